# baseline (speedup 1.0000x reference)
import functools

import jax
import jax.numpy as jnp
from jax import lax
from jax.experimental import pallas as pl
from jax.experimental.pallas import tpu as pltpu
from jax.experimental.pallas import tpu_sc as plsc

_N_OUT = 1000
_IOU_THR = 0.7
_T = 128
_A = 20000
_NB = (_A + _T - 1) // _T
_NBT = _NB * _T
_NV = _NBT // 16
_SEL_PAD = 1024
_NW = 32
_GPAD = 20480
_PER_W = _GPAD // _NW
_CHUNK = 128
_NCH = _PER_W // _CHUNK


def _nms_body(c0_ref, c1_ref, c2_ref, c3_ref,
              keep_ref, cnt_ref,
              area_ref, s_ref, sup_ref, alive_ref, ident_ref, conv_ref):
    NB = c0_ref.shape[0]
    T = _T

    ia = jax.lax.broadcasted_iota(jnp.int32, (T, T), 0)
    ib = jax.lax.broadcasted_iota(jnp.int32, (T, T), 1)
    ident_ref[...] = jnp.where(ia == ib, 1.0, 0.0)
    cnt_ref[0, 0] = 0

    def init_body(bi, _):
        r0 = c0_ref[pl.ds(bi, 1)].reshape(1, T)
        r1 = c1_ref[pl.ds(bi, 1)].reshape(1, T)
        r2 = c2_ref[pl.ds(bi, 1)].reshape(1, T)
        r3 = c3_ref[pl.ds(bi, 1)].reshape(1, T)
        area_ref[pl.ds(bi, 1)] = ((r3 - r1) * (r2 - r0)).reshape(1, 1, T)
        keep_ref[pl.ds(bi, 1)] = jnp.zeros((1, 1, T), jnp.float32)
        return 0

    jax.lax.fori_loop(0, NB, init_body, 0)

    ident = ident_ref[...]

    def _to_col(row):
        return jax.lax.dot_general(ident, row, (((1,), (1,)), ((), ())),
                                   preferred_element_type=jnp.float32)

    def _iou_mask(cols, col_area, rows, row_area):
        s0, s1, s2, s3 = cols
        r0, r1, r2, r3 = rows
        xx1 = jnp.maximum(s1, r1)
        yy1 = jnp.maximum(s0, r0)
        xx2 = jnp.minimum(s3, r3)
        yy2 = jnp.minimum(s2, r2)
        w = jnp.maximum(xx2 - xx1, 0.0)
        h = jnp.maximum(yy2 - yy1, 0.0)
        inter = w * h
        iou = inter / (col_area + row_area - inter)
        return jnp.where(iou > _IOU_THR, 1.0, 0.0)

    def block_body(bi, _):
        @pl.when(cnt_ref[0, 0] < _N_OUT)
        def _():
            r0 = c0_ref[pl.ds(bi, 1)].reshape(1, T)
            r1 = c1_ref[pl.ds(bi, 1)].reshape(1, T)
            r2 = c2_ref[pl.ds(bi, 1)].reshape(1, T)
            r3 = c3_ref[pl.ds(bi, 1)].reshape(1, T)
            r_area = area_ref[pl.ds(bi, 1)].reshape(1, T)

            sup_ref[...] = jnp.zeros((1, T), jnp.float32)

            def cross_body(bj, _c):
                s_rows = (c0_ref[pl.ds(bj, 1)].reshape(1, T),
                          c1_ref[pl.ds(bj, 1)].reshape(1, T),
                          c2_ref[pl.ds(bj, 1)].reshape(1, T),
                          c3_ref[pl.ds(bj, 1)].reshape(1, T))
                cols = tuple(_to_col(r) for r in s_rows)
                c_area = _to_col(area_ref[pl.ds(bj, 1)].reshape(1, T))
                m = _iou_mask(cols, c_area, (r0, r1, r2, r3), r_area)
                kr = keep_ref[pl.ds(bj, 1)].reshape(1, T)
                dead = jax.lax.dot_general(kr, m, (((1,), (0,)), ((), ())),
                                           preferred_element_type=jnp.float32)
                sup_ref[...] = jnp.maximum(sup_ref[...],
                                           jnp.where(dead > 0.0, 1.0, 0.0))
                return 0

            jax.lax.fori_loop(0, bi, cross_body, 0)

            cols = (_to_col(r0), _to_col(r1), _to_col(r2), _to_col(r3))
            c_area = _to_col(r_area)
            m = _iou_mask(cols, c_area, (r0, r1, r2, r3), r_area)
            s_ref[...] = m * jnp.where(ia < ib, 1.0, 0.0)

            lane = jax.lax.broadcasted_iota(jnp.int32, (1, T), 1)
            valid = jnp.where(bi * T + lane < _A, 1.0, 0.0)
            alive0 = (1.0 - sup_ref[...]) * valid
            alive_ref[...] = alive0
            conv_ref[0] = 0

            def fix_body(t, _f):
                @pl.when(conv_ref[0] == 0)
                def _():
                    alive = alive_ref[...]
                    dead = jax.lax.dot_general(
                        alive, s_ref[...], (((1,), (0,)), ((), ())),
                        preferred_element_type=jnp.float32)
                    new_alive = alive0 * jnp.where(dead > 0.0, 0.0, 1.0)
                    changed = jnp.sum(jnp.abs(new_alive - alive))
                    alive_ref[...] = new_alive
                    conv_ref[0] = jnp.where(changed > 0.0, 0, 1)
                return 0

            jax.lax.fori_loop(0, T, fix_body, 0)

            alive = alive_ref[...]
            keep_ref[pl.ds(bi, 1)] = alive.reshape(1, 1, T)
            cnt_ref[0, 0] = cnt_ref[0, 0] + jnp.sum(alive).astype(jnp.int32)
        return 0

    jax.lax.fori_loop(0, NB, block_body, 0)


def _sc_select_body(keep_hbm, cnt_hbm, c0_hbm, c1_hbm, c2_hbm, c3_hbm, sc_hbm,
                    o0_hbm, o1_hbm, o2_hbm, o3_hbm, os_hbm,
                    keep_v, cnt_v, c0_v, c1_v, c2_v, c3_v, sc_v,
                    sel_v, o0_v, o1_v, o2_v, o3_v, os_v):
    @pl.when(jnp.logical_and(lax.axis_index("c") == 0, lax.axis_index("s") == 0))
    def _():
        pltpu.sync_copy(keep_hbm, keep_v)
        pltpu.sync_copy(cnt_hbm, cnt_v)
        pltpu.sync_copy(c0_hbm, c0_v)
        pltpu.sync_copy(c1_hbm, c1_v)
        pltpu.sync_copy(c2_hbm, c2_v)
        pltpu.sync_copy(c3_hbm, c3_v)
        pltpu.sync_copy(sc_hbm, sc_v)

        iota = lax.iota(jnp.int32, 16)
        lv = cnt_v[...]

        def init_body(j, _):
            g = j * 16 + iota
            sel_v[pl.ds(j * 16, 16)] = jnp.maximum(g - lv, 0)
            return 0

        lax.fori_loop(0, _SEL_PAD // 16, init_body, 0)

        def body(g, total):
            for u in range(8):
                i = g * 8 + u
                k = keep_v[pl.ds(i * 16, 16)]
                incl = plsc.cumsum(k)
                rank = incl - k + total
                mask = jnp.logical_and(k > 0, rank < _N_OUT)
                plsc.store_scatter(sel_v, [rank], i * 16 + iota, mask=mask)
                total = total + plsc.all_reduce_population_count(k > 0)
            return total

        lax.fori_loop(0, _NV // 8, body, jnp.zeros((16,), jnp.int32))

        def gather_body(j, _):
            idx = sel_v[pl.ds(j * 16, 16)]
            o0_v[pl.ds(j * 16, 16)] = plsc.load_gather(c0_v, [idx])
            o1_v[pl.ds(j * 16, 16)] = plsc.load_gather(c1_v, [idx])
            o2_v[pl.ds(j * 16, 16)] = plsc.load_gather(c2_v, [idx])
            o3_v[pl.ds(j * 16, 16)] = plsc.load_gather(c3_v, [idx])
            os_v[pl.ds(j * 16, 16)] = plsc.load_gather(sc_v, [idx])
            return 0

        lax.fori_loop(0, _SEL_PAD // 16, gather_body, 0)

        pltpu.sync_copy(o0_v, o0_hbm)
        pltpu.sync_copy(o1_v, o1_hbm)
        pltpu.sync_copy(o2_v, o2_hbm)
        pltpu.sync_copy(o3_v, o3_hbm)
        pltpu.sync_copy(os_v, os_hbm)


def _sc_select(keep_i, cnt_vec, c0, c1, c2, c3, sc):
    f32, i32 = jnp.float32, jnp.int32
    k = pl.kernel(
        _sc_select_body,
        mesh=plsc.VectorSubcoreMesh(core_axis_name="c", subcore_axis_name="s"),
        compiler_params=pltpu.CompilerParams(needs_layout_passes=False),
        out_type=[jax.ShapeDtypeStruct((_SEL_PAD,), f32)] * 5,
        scratch_types=[
            pltpu.VMEM((_NBT,), i32),
            pltpu.VMEM((16,), i32),
            pltpu.VMEM((_NBT,), f32),
            pltpu.VMEM((_NBT,), f32),
            pltpu.VMEM((_NBT,), f32),
            pltpu.VMEM((_NBT,), f32),
            pltpu.VMEM((_NBT,), f32),
            pltpu.VMEM((_SEL_PAD,), i32),
            pltpu.VMEM((_SEL_PAD,), f32),
            pltpu.VMEM((_SEL_PAD,), f32),
            pltpu.VMEM((_SEL_PAD,), f32),
            pltpu.VMEM((_SEL_PAD,), f32),
            pltpu.VMEM((_SEL_PAD,), f32),
        ],
    )
    return k(keep_i, cnt_vec, c0, c1, c2, c3, sc)


_NBINS = 256
_SNV = _GPAD // 16


def _sc_radix_body(sc_hbm, order_hbm,
                   k0_v, i0_v, k1_v, i1_v, hist_v, sf_v):
    @pl.when(jnp.logical_and(lax.axis_index("c") == 0, lax.axis_index("s") == 0))
    def _():
        iota = lax.iota(jnp.int32, 16)
        ones = jnp.ones((16,), jnp.int32)

        pltpu.sync_copy(sc_hbm, sf_v)

        def key_body(i, _):
            s = sf_v[pl.ds(i * 16, 16)]
            u = plsc.bitcast(s, jnp.int32)
            neg = u < 0
            k = jnp.where(neg, u, jnp.bitwise_and(~u, jnp.int32(0x7FFFFFFF)))
            g = i * 16 + iota
            k = jnp.where(g < _A, k, jnp.int32(-1))
            k0_v[pl.ds(i * 16, 16)] = k
            i0_v[pl.ds(i * 16, 16)] = jnp.where(g < _A, g, 0)
            return 0

        lax.fori_loop(0, _SNV, key_body, 0)

        def make_pass(src_k, src_i, dst_k, dst_i, shift):
            def hist_clear(i, _):
                hist_v[pl.ds(i * 16, 16)] = jnp.zeros((16,), jnp.int32)
                return 0

            lax.fori_loop(0, _NBINS // 16, hist_clear, 0)

            def hist_body(i, _):
                k = src_k[pl.ds(i * 16, 16)]
                d = jnp.bitwise_and(lax.shift_right_logical(k, shift), 255)
                plsc.addupdate_scatter(hist_v, [d], ones)
                return 0

            lax.fori_loop(0, _SNV, hist_body, 0)

            def pref_body(i, total):
                h = hist_v[pl.ds(i * 16, 16)]
                incl = plsc.cumsum(h)
                hist_v[pl.ds(i * 16, 16)] = incl - h + total
                return total + jnp.sum(h)

            lax.fori_loop(0, _NBINS // 16, pref_body, jnp.int32(0))

            def scat_body(i, _):
                k = src_k[pl.ds(i * 16, 16)]
                idx = src_i[pl.ds(i * 16, 16)]
                d = jnp.bitwise_and(lax.shift_right_logical(k, shift), 255)
                base = plsc.load_gather(hist_v, [d])
                occ, _last = plsc.scan_count(d)
                rank = base + occ - 1
                plsc.store_scatter(dst_k, [rank], k)
                plsc.store_scatter(dst_i, [rank], idx)
                plsc.addupdate_scatter(hist_v, [d], ones)
                return 0

            lax.fori_loop(0, _SNV, scat_body, 0)

        make_pass(k0_v, i0_v, k1_v, i1_v, 0)
        make_pass(k1_v, i1_v, k0_v, i0_v, 8)
        make_pass(k0_v, i0_v, k1_v, i1_v, 16)
        make_pass(k1_v, i1_v, k0_v, i0_v, 24)

        pltpu.sync_copy(i0_v, order_hbm)


def _sc_radix_argsort(scores):
    i32, f32 = jnp.int32, jnp.float32
    k = pl.kernel(
        _sc_radix_body,
        mesh=plsc.VectorSubcoreMesh(core_axis_name="c", subcore_axis_name="s"),
        compiler_params=pltpu.CompilerParams(needs_layout_passes=False),
        out_type=[jax.ShapeDtypeStruct((_GPAD,), i32)],
        scratch_types=[
            pltpu.VMEM((_GPAD,), i32),
            pltpu.VMEM((_GPAD,), i32),
            pltpu.VMEM((_GPAD,), i32),
            pltpu.VMEM((_GPAD,), i32),
            pltpu.VMEM((_NBINS,), i32),
            pltpu.VMEM((_GPAD,), f32),
        ],
    )
    (order,) = k(scores)
    return order


def _sc_pregather_body(order_hbm, b0_hbm, b1_hbm, b2_hbm, b3_hbm, sc_hbm,
                       o0_hbm, o1_hbm, o2_hbm, o3_hbm, os_hbm,
                       idx_v, g0_v, g1_v, g2_v, g3_v, g4_v, sem):
    wid = lax.axis_index("s") * 2 + lax.axis_index("c")
    base = wid * _PER_W
    pltpu.sync_copy(order_hbm.at[pl.ds(base, _PER_W)], idx_v)
    srcs = (b0_hbm, b1_hbm, b2_hbm, b3_hbm, sc_hbm)
    dsts = (o0_hbm, o1_hbm, o2_hbm, o3_hbm, os_hbm)
    gs = (g0_v, g1_v, g2_v, g3_v, g4_v)
    copies = []
    for a in range(5):
        for ch in range(_NCH):
            copies.append(pltpu.async_copy(
                srcs[a].at[idx_v.at[pl.ds(ch * _CHUNK, _CHUNK)]],
                gs[a].at[pl.ds(ch * _CHUNK, _CHUNK)], sem))
    for c in copies:
        c.wait()
    for a in range(5):
        pltpu.sync_copy(gs[a], dsts[a].at[pl.ds(base, _PER_W)])


def _sc_pregather(order_pad, b0, b1, b2, b3, sc):
    f32, i32 = jnp.float32, jnp.int32
    k = pl.kernel(
        _sc_pregather_body,
        mesh=plsc.VectorSubcoreMesh(core_axis_name="c", subcore_axis_name="s"),
        compiler_params=pltpu.CompilerParams(needs_layout_passes=False),
        out_type=[jax.ShapeDtypeStruct((_GPAD,), f32)] * 5,
        scratch_types=[
            pltpu.VMEM((_PER_W,), i32),
            pltpu.VMEM((_PER_W,), f32),
            pltpu.VMEM((_PER_W,), f32),
            pltpu.VMEM((_PER_W,), f32),
            pltpu.VMEM((_PER_W,), f32),
            pltpu.VMEM((_PER_W,), f32),
            pltpu.SemaphoreType.DMA,
        ],
    )
    return k(order_pad, b0, b1, b2, b3, sc)


@functools.partial(jax.jit, static_argnums=())
def kernel(rpn_reg_absolute, rpn_cls):
    B, A = rpn_cls.shape
    boxes = rpn_reg_absolute[0]
    scores = rpn_cls[0]

    order_pad = _sc_radix_argsort(jnp.pad(scores, (0, _GPAD - A)))

    c0s, c1s, c2s, c3s, scs = _sc_pregather(
        order_pad, boxes[:, 0], boxes[:, 1], boxes[:, 2], boxes[:, 3], scores)

    NB = _NB
    crows = [c[:_NBT].reshape(NB, 1, _T) for c in (c0s, c1s, c2s, c3s)]

    keep2d, cnt = pl.pallas_call(
        _nms_body,
        out_shape=[
            jax.ShapeDtypeStruct((NB, 1, _T), jnp.float32),
            jax.ShapeDtypeStruct((1, 1), jnp.int32),
        ],
        out_specs=[
            pl.BlockSpec(memory_space=pltpu.VMEM),
            pl.BlockSpec(memory_space=pltpu.SMEM),
        ],
        in_specs=[pl.BlockSpec(memory_space=pltpu.VMEM)] * 4,
        scratch_shapes=[
            pltpu.VMEM((NB, 1, _T), jnp.float32),
            pltpu.VMEM((_T, _T), jnp.float32),
            pltpu.VMEM((1, _T), jnp.float32),
            pltpu.VMEM((1, _T), jnp.float32),
            pltpu.VMEM((_T, _T), jnp.float32),
            pltpu.SMEM((1,), jnp.int32),
        ],
    )(*crows)

    keep_i = (keep2d.reshape(_NBT) > 0.5).astype(jnp.int32)
    cnt_vec = jnp.full((16,), cnt[0, 0], jnp.int32)

    o0, o1, o2, o3, osc = _sc_select(
        keep_i, cnt_vec, c0s[:_NBT], c1s[:_NBT], c2s[:_NBT], c3s[:_NBT],
        scs[:_NBT])

    out_boxes = jnp.stack(
        [o0[:_N_OUT], o1[:_N_OUT], o2[:_N_OUT], o3[:_N_OUT]], axis=-1)[None]
    out_scores = osc[:_N_OUT][None]
    return (out_boxes, out_scores)
